# baseline (speedup 1.0000x reference)
import functools

import jax
import jax.numpy as jnp
from jax import lax
from jax.experimental import pallas as pl
from jax.experimental.pallas import tpu as pltpu
from jax.experimental.pallas import tpu_sc as plsc

_D = 1024
_B = 16384
_NC = 2
_NS = 16
_NW = _NC * _NS
_BPW = _B // _NW
_SCALE = 32.0

_SIZES = (64, 56, 64, 56, 64, 56, 64, 56, 32)
_OFFS = tuple(sum(_SIZES[:i]) for i in range(len(_SIZES)))
_NCHUNK = len(_SIZES)

_mesh = plsc.VectorSubcoreMesh(core_axis_name="c", subcore_axis_name="s")


@functools.partial(
    pl.kernel,
    out_type=jax.ShapeDtypeStruct((_B, _D), jnp.float32),
    mesh=_mesh,
    scratch_types=[
        pltpu.VMEM((_BPW,), jnp.int32),
        pltpu.VMEM((64, _D), jnp.float32),
        pltpu.VMEM((56, _D), jnp.float32),
    ]
    + [pltpu.SemaphoreType.DMA] * 4,
)
def _embed_sc(table_hbm, idx_hbm, out_hbm, idx_v, rows0_v, rows1_v, *sems):
    rows = (rows0_v, rows1_v)
    gsem = sems[:2]
    osem = sems[2:]
    wid = lax.axis_index("s") * _NC + lax.axis_index("c")
    base = wid * _BPW
    pltpu.sync_copy(idx_hbm.at[pl.ds(base, _BPW)], idx_v)

    def gather(j):
        b, n = j % 2, _SIZES[j]
        return pltpu.make_async_copy(
            table_hbm.at[idx_v.at[pl.ds(_OFFS[j], n)]],
            rows[b].at[pl.ds(0, n)],
            gsem[b],
        )

    def writeback(j):
        b, n = j % 2, _SIZES[j]
        return pltpu.make_async_copy(
            rows[b].at[pl.ds(0, n)],
            out_hbm.at[pl.ds(base + _OFFS[j], n)],
            osem[b],
        )

    gather(0).start()

    for j in range(_NCHUNK):
        b = j % 2
        gather(j).wait()
        if j + 1 < _NCHUNK:
            if j >= 1:
                writeback(j - 1).wait()
            gather(j + 1).start()

        @plsc.parallel_loop(0, _SIZES[j])
        def _row(r, b=b):
            @plsc.parallel_loop(0, _D // 16, unroll=8)
            def _col(c, r=r, b=b):
                sl = pl.ds(c * 16, 16)
                rows[b][r, sl] = rows[b][r, sl] * _SCALE

        writeback(j).start()

    writeback(_NCHUNK - 2).wait()
    writeback(_NCHUNK - 1).wait()


@jax.jit
def kernel(x, input_embedding_table):
    idx = x.reshape(_B).astype(jnp.int32)
    out = _embed_sc(input_embedding_table, idx)
    return out.reshape(x.shape + (_D,))


# speedup vs baseline: 1.0115x; 1.0115x over previous
import functools

import jax
import jax.numpy as jnp
from jax import lax
from jax.experimental import pallas as pl
from jax.experimental.pallas import tpu as pltpu
from jax.experimental.pallas import tpu_sc as plsc

_D = 1024
_B = 16384
_NC = 2
_NS = 16
_NW = _NC * _NS
_BPW = _B // _NW
_SCALE = 32.0
_PF = 2
_NBUF = 3

_SIZES = (48, 40, 32) * 4 + (32,)
_OFFS = tuple(sum(_SIZES[:i]) for i in range(len(_SIZES)))
_NCHUNK = len(_SIZES)
_BUFROWS = (48, 40, 32)

_mesh = plsc.VectorSubcoreMesh(core_axis_name="c", subcore_axis_name="s")


@functools.partial(
    pl.kernel,
    out_type=jax.ShapeDtypeStruct((_B, _D), jnp.float32),
    mesh=_mesh,
    scratch_types=[
        pltpu.VMEM((_BPW,), jnp.int32),
        pltpu.VMEM((_BUFROWS[0], _D), jnp.float32),
        pltpu.VMEM((_BUFROWS[1], _D), jnp.float32),
        pltpu.VMEM((_BUFROWS[2], _D), jnp.float32),
    ]
    + [pltpu.SemaphoreType.DMA] * (2 * _NBUF),
)
def _embed_sc(table_hbm, idx_hbm, out_hbm, idx_v, r0, r1, r2, *sems):
    rows = (r0, r1, r2)
    gsem = sems[:_NBUF]
    osem = sems[_NBUF:]
    wid = lax.axis_index("s") * _NC + lax.axis_index("c")
    base = wid * _BPW
    pltpu.sync_copy(idx_hbm.at[pl.ds(base, _BPW)], idx_v)

    def gather(j):
        b, n = j % _NBUF, _SIZES[j]
        return pltpu.make_async_copy(
            table_hbm.at[idx_v.at[pl.ds(_OFFS[j], n)]],
            rows[b].at[pl.ds(0, n)],
            gsem[b],
        )

    def writeback(j):
        b, n = j % _NBUF, _SIZES[j]
        return pltpu.make_async_copy(
            rows[b].at[pl.ds(0, n)],
            out_hbm.at[pl.ds(base + _OFFS[j], n)],
            osem[b],
        )

    for j in range(_PF):
        gather(j).start()

    for j in range(_NCHUNK):
        b = j % _NBUF
        gather(j).wait()
        jn = j + _PF
        if jn < _NCHUNK:
            if jn - _NBUF >= 0:
                writeback(jn - _NBUF).wait()
            gather(jn).start()

        @plsc.parallel_loop(0, _SIZES[j])
        def _row(r, b=b):
            @plsc.parallel_loop(0, _D // 16, unroll=8)
            def _col(c, r=r, b=b):
                sl = pl.ds(c * 16, 16)
                rows[b][r, sl] = rows[b][r, sl] * _SCALE

        writeback(j).start()

    for j in range(_NCHUNK - _NBUF, _NCHUNK):
        writeback(j).wait()


@jax.jit
def kernel(x, input_embedding_table):
    idx = x.reshape(_B).astype(jnp.int32)
    out = _embed_sc(input_embedding_table, idx)
    return out.reshape(x.shape + (_D,))
